# Initial kernel scaffold; baseline (speedup 1.0000x reference)
import jax
import jax.numpy as jnp
from jax.experimental import pallas as pl


def kernel(q, paged_kv_cache, kv_page_indptr, kv_page_indices, sparse_ind, sparse_nnz):
    raise NotImplementedError("write your pallas kernel here")


# speedup vs baseline: 242.6581x; 242.6581x over previous
import functools

import jax
import jax.numpy as jnp
from jax import lax
from jax.experimental import pallas as pl
from jax.experimental.pallas import tpu as pltpu
from jax.experimental.pallas import tpu_sc as plsc

B, H, KVH, D = 16, 32, 8, 128
G = H // KVH
PAGE = 16
KVLEN = 4096
NPAGES = KVLEN // PAGE
L = 2048
NW = 32
ROWS = B * H
RPW = ROWS // NW
LANES = 16

@functools.cache
def _sc_mesh():
    return plsc.VectorSubcoreMesh(
        core_axis_name="c", subcore_axis_name="s", num_cores=2, num_subcores=16)


def _hist_body(ind_hbm, nnz_hbm, out_hbm, idx0, idx1, hist_v, nnz_v, sem0, sem1):
    wid = lax.axis_index("s") * 2 + lax.axis_index("c")
    base_row = wid * RPW
    pltpu.sync_copy(nnz_hbm.at[pl.ds(base_row, RPW)], nnz_v.at[pl.ds(0, RPW)])
    pos16 = lax.iota(jnp.int32, LANES)
    ones16 = jnp.ones((LANES,), jnp.float32)
    zeros16 = jnp.zeros((LANES,), jnp.float32)

    def zero_body(c, carry):
        hist_v[pl.ds(c * LANES, LANES)] = zeros16
        return carry

    lax.fori_loop(0, KVLEN // LANES, zero_body, 0, unroll=8)

    pltpu.make_async_copy(ind_hbm.at[base_row], idx0, sem0).start()

    UN = 4

    def process(idx_v, row, nnz_s):
        nblk = (nnz_s + (LANES * UN - 1)) // (LANES * UN)

        def scat_blk(bk, carry2):
            for u in range(UN):
                c = bk * UN + u
                idx16 = idx_v[pl.ds(c * LANES, LANES)]
                mask = (c * LANES + pos16) < nnz_s
                plsc.addupdate_scatter(hist_v, [idx16], ones16, mask=mask)
            return carry2

        lax.fori_loop(0, nblk, scat_blk, 0)
        pltpu.sync_copy(hist_v, out_hbm.at[row])

        def clear_blk(bk, carry2):
            for u in range(UN):
                c = bk * UN + u
                idx16 = idx_v[pl.ds(c * LANES, LANES)]
                mask = (c * LANES + pos16) < nnz_s
                plsc.store_scatter(hist_v, [idx16], zeros16, mask=mask)
            return carry2

        lax.fori_loop(0, nblk, clear_blk, 0)

    def pair_body(i, carry):
        r0 = 2 * i
        r1 = 2 * i + 1
        pltpu.make_async_copy(ind_hbm.at[base_row], idx0, sem0).wait()
        pltpu.make_async_copy(ind_hbm.at[base_row + r1], idx1, sem1).start()
        process(idx0, base_row + r0, nnz_v[pl.ds(r0, LANES)][0])
        pltpu.make_async_copy(ind_hbm.at[base_row + r1], idx1, sem1).wait()

        @pl.when(r1 + 1 < RPW)
        def _():
            pltpu.make_async_copy(ind_hbm.at[base_row + r1 + 1], idx0, sem0).start()

        process(idx1, base_row + r1, nnz_v[pl.ds(r1, LANES)][0])
        return carry

    lax.fori_loop(0, RPW // 2, pair_body, 0)


def _histogram(ind2, nnz1, interpret=False):
    return pl.kernel(
        _hist_body,
        out_type=jax.ShapeDtypeStruct((ROWS, KVLEN), jnp.float32),
        mesh=_sc_mesh(),
        scratch_types=[
            pltpu.VMEM((L,), jnp.int32),
            pltpu.VMEM((L,), jnp.int32),
            pltpu.VMEM((KVLEN,), jnp.float32),
            pltpu.VMEM((RPW + LANES,), jnp.int32),
            pltpu.SemaphoreType.DMA,
            pltpu.SemaphoreType.DMA,
        ],
        compiler_params=pltpu.CompilerParams(needs_layout_passes=False),
        interpret=interpret,
    )(ind2, nnz1)


HPAGES = NPAGES // 2
HTOK = KVLEN // 2


NBUF = 3


def _attn_body(indptr_s, pages_s, q_ref, c_ref, cache_ref, out_ref,
               bufs, o_scr, m_scr, l_scr, sems):
    b = pl.program_id(0)
    scale = jnp.float32(1.0 / (D ** 0.5))

    def issue_half(ch):
        bb = ch // 2
        p0 = (ch % 2) * HPAGES
        slot = lax.rem(ch, NBUF)
        base = indptr_s[bb]

        def issue(j, carry):
            page = pages_s[base + p0 + j]
            pltpu.make_async_copy(
                cache_ref.at[page], bufs.at[slot, j], sems.at[slot]).start()
            return carry

        lax.fori_loop(0, HPAGES, issue, 0, unroll=8)

    def drain_half(slot):
        def drain(j, carry):
            pltpu.make_async_copy(
                cache_ref.at[0], bufs.at[slot, 0], sems.at[slot]).wait()
            return carry

        lax.fori_loop(0, HPAGES, drain, 0, unroll=8)

    @pl.when(b == 0)
    def _():
        for ch in range(NBUF):
            issue_half(ch)

    def half_stats(slot, kvh, t0):
        k = bufs[slot, :, 0, kvh].reshape(HTOK, D)
        v = bufs[slot, :, 1, kvh].reshape(HTOK, D)
        qg = q_ref[0, pl.ds(kvh * G, G), 0, :]
        cg = c_ref[0, pl.ds(kvh * G, G), pl.ds(t0, HTOK)]
        s = lax.dot_general(qg, k, (((1,), (1,)), ((), ())),
                            preferred_element_type=jnp.float32) * scale
        s = jnp.where(cg > 0.0, s, jnp.float32(-1e6))
        m = jnp.max(s, axis=1, keepdims=True)
        w = cg * jnp.exp(s - m)
        l = jnp.sum(w, axis=1, keepdims=True)
        o = lax.dot_general(w, v, (((1,), (0,)), ((), ())),
                            preferred_element_type=jnp.float32)
        return m, l, o

    slot_a = lax.rem(2 * b, NBUF)
    slot_b = lax.rem(2 * b + 1, NBUF)

    drain_half(slot_a)
    for kvh in range(KVH):
        m1, l1, o1 = half_stats(slot_a, kvh, 0)
        o_scr[kvh] = o1
        m_scr[kvh] = m1
        l_scr[kvh] = l1

    @pl.when(2 * b + NBUF < 2 * B)
    def _():
        issue_half(2 * b + NBUF)

    drain_half(slot_b)
    for kvh in range(KVH):
        m2, l2, o2 = half_stats(slot_b, kvh, HTOK)
        m1 = m_scr[kvh]
        l1 = l_scr[kvh]
        o1 = o_scr[kvh]
        mm = jnp.maximum(m1, m2)
        a1 = jnp.exp(m1 - mm)
        a2 = jnp.exp(m2 - mm)
        o = o1 * a1 + o2 * a2
        l = l1 * a1 + l2 * a2
        out_ref[0, pl.ds(kvh * G, G), 0, :] = o / l

    @pl.when(2 * b + NBUF + 1 < 2 * B)
    def _():
        issue_half(2 * b + NBUF + 1)


def _attention(indptr, pages, q, counts3, cache, interpret=False):
    grid_spec = pltpu.PrefetchScalarGridSpec(
        num_scalar_prefetch=2,
        grid=(B,),
        in_specs=[
            pl.BlockSpec((1, H, 1, D), lambda b, *_: (b, 0, 0, 0)),
            pl.BlockSpec((1, H, KVLEN), lambda b, *_: (b, 0, 0)),
            pl.BlockSpec(memory_space=pl.ANY),
        ],
        out_specs=pl.BlockSpec((1, H, 1, D), lambda b, *_: (b, 0, 0, 0)),
        scratch_shapes=[
            pltpu.VMEM((NBUF, HPAGES, 2, KVH, PAGE, D), jnp.float32),
            pltpu.VMEM((KVH, G, D), jnp.float32),
            pltpu.VMEM((KVH, G, 1), jnp.float32),
            pltpu.VMEM((KVH, G, 1), jnp.float32),
            pltpu.SemaphoreType.DMA((NBUF,)),
        ],
    )
    return pl.pallas_call(
        _attn_body,
        grid_spec=grid_spec,
        out_shape=jax.ShapeDtypeStruct((B, H, 1, D), jnp.float32),
        interpret=interpret,
    )(indptr, pages, q, counts3, cache)


def kernel(q, paged_kv_cache, kv_page_indptr, kv_page_indices, sparse_ind, sparse_nnz):
    counts = _histogram(sparse_ind.reshape(ROWS, L), sparse_nnz.reshape(ROWS))
    counts3 = counts.reshape(B, H, KVLEN)
    return _attention(kv_page_indptr, kv_page_indices, q, counts3, paged_kv_cache)


# speedup vs baseline: 253.0508x; 1.0428x over previous
import functools

import jax
import jax.numpy as jnp
from jax import lax
from jax.experimental import pallas as pl
from jax.experimental.pallas import tpu as pltpu
from jax.experimental.pallas import tpu_sc as plsc

B, H, KVH, D = 16, 32, 8, 128
G = H // KVH
PAGE = 16
KVLEN = 4096
NPAGES = KVLEN // PAGE
L = 2048
NW = 32
ROWS = B * H
RPW = ROWS // NW
LANES = 16

@functools.cache
def _sc_mesh():
    return plsc.VectorSubcoreMesh(
        core_axis_name="c", subcore_axis_name="s", num_cores=2, num_subcores=16)


def _hist_body(ind_hbm, nnz_hbm, out_hbm, idx0, idx1, hist_v, nnz_v, sem0, sem1):
    wid = lax.axis_index("s") * 2 + lax.axis_index("c")
    base_row = wid * RPW
    pltpu.sync_copy(nnz_hbm.at[pl.ds(base_row, RPW)], nnz_v.at[pl.ds(0, RPW)])
    pos16 = lax.iota(jnp.int32, LANES)
    ones16 = jnp.ones((LANES,), jnp.float32)
    zeros16 = jnp.zeros((LANES,), jnp.float32)

    def zero_body(c, carry):
        hist_v[pl.ds(c * LANES, LANES)] = zeros16
        return carry

    lax.fori_loop(0, KVLEN // LANES, zero_body, 0, unroll=8)

    pltpu.make_async_copy(ind_hbm.at[base_row], idx0, sem0).start()

    UN = 4

    def process(idx_v, row, nnz_s):
        nblk = (nnz_s + (LANES * UN - 1)) // (LANES * UN)

        def scat_blk(bk, carry2):
            for u in range(UN):
                c = bk * UN + u
                idx16 = idx_v[pl.ds(c * LANES, LANES)]
                mask = (c * LANES + pos16) < nnz_s
                plsc.addupdate_scatter(hist_v, [idx16], ones16, mask=mask)
            return carry2

        lax.fori_loop(0, nblk, scat_blk, 0)
        pltpu.sync_copy(hist_v, out_hbm.at[row])

        def clear_blk(bk, carry2):
            for u in range(UN):
                c = bk * UN + u
                idx16 = idx_v[pl.ds(c * LANES, LANES)]
                mask = (c * LANES + pos16) < nnz_s
                plsc.store_scatter(hist_v, [idx16], zeros16, mask=mask)
            return carry2

        lax.fori_loop(0, nblk, clear_blk, 0)

    def pair_body(i, carry):
        r0 = 2 * i
        r1 = 2 * i + 1
        pltpu.make_async_copy(ind_hbm.at[base_row], idx0, sem0).wait()
        pltpu.make_async_copy(ind_hbm.at[base_row + r1], idx1, sem1).start()
        process(idx0, base_row + r0, nnz_v[pl.ds(r0, LANES)][0])
        pltpu.make_async_copy(ind_hbm.at[base_row + r1], idx1, sem1).wait()

        @pl.when(r1 + 1 < RPW)
        def _():
            pltpu.make_async_copy(ind_hbm.at[base_row + r1 + 1], idx0, sem0).start()

        process(idx1, base_row + r1, nnz_v[pl.ds(r1, LANES)][0])
        return carry

    lax.fori_loop(0, RPW // 2, pair_body, 0)


def _histogram(ind2, nnz1):
    return pl.kernel(
        _hist_body,
        out_type=jax.ShapeDtypeStruct((ROWS, KVLEN), jnp.float32),
        mesh=_sc_mesh(),
        scratch_types=[
            pltpu.VMEM((L,), jnp.int32),
            pltpu.VMEM((L,), jnp.int32),
            pltpu.VMEM((KVLEN,), jnp.float32),
            pltpu.VMEM((RPW + LANES,), jnp.int32),
            pltpu.SemaphoreType.DMA,
            pltpu.SemaphoreType.DMA,
        ],
        compiler_params=pltpu.CompilerParams(needs_layout_passes=False),
    )(ind2, nnz1)


HPAGES = NPAGES // 2
HTOK = KVLEN // 2


NBUF = 3


def _attn_body(indptr_s, pages_s, q_ref, c_ref, cache_ref, out_ref,
               bufs, o_scr, m_scr, l_scr, sems):
    b = pl.program_id(0)
    scale = jnp.float32(1.0 / (D ** 0.5))

    def issue_half(ch):
        bb = ch // 2
        p0 = (ch % 2) * HPAGES
        slot = lax.rem(ch, NBUF)
        base = indptr_s[bb]

        def issue(j, carry):
            page = pages_s[base + p0 + j]
            pltpu.make_async_copy(
                cache_ref.at[page], bufs.at[slot, j], sems.at[slot]).start()
            return carry

        lax.fori_loop(0, HPAGES, issue, 0, unroll=8)

    def drain_half(slot):
        def drain(j, carry):
            pltpu.make_async_copy(
                cache_ref.at[0], bufs.at[slot, 0], sems.at[slot]).wait()
            return carry

        lax.fori_loop(0, HPAGES, drain, 0, unroll=8)

    @pl.when(b == 0)
    def _():
        for ch in range(NBUF):
            issue_half(ch)

    def half_stats(slot, kvh, t0):
        k = bufs[slot, :, 0, kvh].reshape(HTOK, D)
        v = bufs[slot, :, 1, kvh].reshape(HTOK, D)
        qg = q_ref[0, pl.ds(kvh * G, G), 0, :]
        cg = c_ref[0, pl.ds(kvh * G, G), pl.ds(t0, HTOK)]
        s = lax.dot_general(qg, k, (((1,), (1,)), ((), ())),
                            preferred_element_type=jnp.float32) * scale
        s = jnp.where(cg > 0.0, s, jnp.float32(-1e6))
        m = jnp.max(s, axis=1, keepdims=True)
        w = cg * jnp.exp(s - m)
        l = jnp.sum(w, axis=1, keepdims=True)
        o = lax.dot_general(w, v, (((1,), (0,)), ((), ())),
                            preferred_element_type=jnp.float32)
        return m, l, o

    slot_a = lax.rem(2 * b, NBUF)
    slot_b = lax.rem(2 * b + 1, NBUF)

    drain_half(slot_a)
    for kvh in range(KVH):
        m1, l1, o1 = half_stats(slot_a, kvh, 0)
        o_scr[kvh] = o1
        m_scr[kvh] = m1
        l_scr[kvh] = l1

    @pl.when(2 * b + NBUF < 2 * B)
    def _():
        issue_half(2 * b + NBUF)

    drain_half(slot_b)
    for kvh in range(KVH):
        m2, l2, o2 = half_stats(slot_b, kvh, HTOK)
        m1 = m_scr[kvh]
        l1 = l_scr[kvh]
        o1 = o_scr[kvh]
        mm = jnp.maximum(m1, m2)
        a1 = jnp.exp(m1 - mm)
        a2 = jnp.exp(m2 - mm)
        o = o1 * a1 + o2 * a2
        l = l1 * a1 + l2 * a2
        out_ref[0, pl.ds(kvh * G, G), 0, :] = o / l

    @pl.when(2 * b + NBUF + 1 < 2 * B)
    def _():
        issue_half(2 * b + NBUF + 1)


def _attention(indptr, pages, q, counts3, cache):
    grid_spec = pltpu.PrefetchScalarGridSpec(
        num_scalar_prefetch=2,
        grid=(B,),
        in_specs=[
            pl.BlockSpec((1, H, 1, D), lambda b, *_: (b, 0, 0, 0)),
            pl.BlockSpec((1, H, KVLEN), lambda b, *_: (b, 0, 0)),
            pl.BlockSpec(memory_space=pl.ANY),
        ],
        out_specs=pl.BlockSpec((1, H, 1, D), lambda b, *_: (b, 0, 0, 0)),
        scratch_shapes=[
            pltpu.VMEM((NBUF, HPAGES, 2, KVH, PAGE, D), jnp.float32),
            pltpu.VMEM((KVH, G, D), jnp.float32),
            pltpu.VMEM((KVH, G, 1), jnp.float32),
            pltpu.VMEM((KVH, G, 1), jnp.float32),
            pltpu.SemaphoreType.DMA((NBUF,)),
        ],
    )
    return pl.pallas_call(
        _attn_body,
        grid_spec=grid_spec,
        out_shape=jax.ShapeDtypeStruct((B, H, 1, D), jnp.float32),
    )(indptr, pages, q, counts3, cache)


def kernel(q, paged_kv_cache, kv_page_indptr, kv_page_indices, sparse_ind, sparse_nnz):
    counts = _histogram(sparse_ind.reshape(ROWS, L), sparse_nnz.reshape(ROWS))
    counts3 = counts.reshape(B, H, KVLEN)
    return _attention(kv_page_indptr, kv_page_indices, q, counts3, paged_kv_cache)
